# Initial kernel scaffold; baseline (speedup 1.0000x reference)
import jax
import jax.numpy as jnp
from jax.experimental import pallas as pl


def kernel(x, edge_index, pos, W1, b1, W2, b2, Wx, Wy, Ws, Wns, Wg):
    raise NotImplementedError("write your pallas kernel here")


# speedup vs baseline: 2.3778x; 2.3778x over previous
import functools

import jax
import jax.numpy as jnp
import numpy as np
from jax import lax
from jax.experimental import pallas as pl
from jax.experimental.pallas import tpu as pltpu
from jax.experimental.pallas import tpu_sc as plsc

_NC = 2
_NS = 16
_NW = _NC * _NS
_H = jax.lax.Precision.HIGHEST


def _tc_matmul(x, Wx):
    n, d = x.shape
    b = 1000

    def body(x_ref, w_ref, o_ref):
        o_ref[...] = jnp.dot(x_ref[...], w_ref[...], precision=_H)

    return pl.pallas_call(
        body,
        grid=(n // b,),
        in_specs=[
            pl.BlockSpec((b, d), lambda i: (i, 0)),
            pl.BlockSpec(Wx.shape, lambda i: (0, 0)),
        ],
        out_specs=pl.BlockSpec((b, Wx.shape[1]), lambda i: (i, 0)),
        out_shape=jax.ShapeDtypeStruct((n, Wx.shape[1]), jnp.float32),
    )(x, Wx)


def _sc_gather_all(xw, pos16, src, dst):
    e = src.shape[0]
    d = xw.shape[1]
    c = 400
    per_w = e // _NW
    steps = per_w // c
    mesh = plsc.VectorSubcoreMesh(core_axis_name="c", subcore_axis_name="s")

    @functools.partial(
        pl.kernel,
        out_type=(
            jax.ShapeDtypeStruct((e, d), jnp.float32),
            jax.ShapeDtypeStruct((e, 16), jnp.float32),
            jax.ShapeDtypeStruct((e, 16), jnp.float32),
        ),
        mesh=mesh,
        scratch_types=[
            pltpu.VMEM((c,), jnp.int32),
            pltpu.VMEM((c,), jnp.int32),
            pltpu.VMEM((c, d), jnp.float32),
            pltpu.VMEM((c, 16), jnp.float32),
            pltpu.VMEM((c, 16), jnp.float32),
            pltpu.SemaphoreType.DMA,
            pltpu.SemaphoreType.DMA,
            pltpu.SemaphoreType.DMA,
        ],
        compiler_params=pltpu.CompilerParams(use_tc_tiling_on_sc=False),
    )
    def k(xw_hbm, pos_hbm, src_hbm, dst_hbm, xwg_hbm, ps_hbm, pd_hbm,
          idxs_v, idxd_v, rows_v, ps_v, pd_v, sem1, sem2, sem3):
        wid = lax.axis_index("s") * _NC + lax.axis_index("c")
        base = wid * per_w

        @pl.loop(0, steps)
        def _(i):
            off = base + i * c
            pltpu.sync_copy(src_hbm.at[pl.ds(off, c)], idxs_v)
            pltpu.sync_copy(dst_hbm.at[pl.ds(off, c)], idxd_v)
            cp1 = pltpu.async_copy(xw_hbm.at[idxs_v], rows_v, sem1)
            cp2 = pltpu.async_copy(pos_hbm.at[idxs_v], ps_v, sem2)
            cp3 = pltpu.async_copy(pos_hbm.at[idxd_v], pd_v, sem3)
            cp1.wait()
            cp2.wait()
            cp3.wait()
            pltpu.sync_copy(rows_v, xwg_hbm.at[pl.ds(off, c)])
            pltpu.sync_copy(ps_v, ps_hbm.at[pl.ds(off, c)])
            pltpu.sync_copy(pd_v, pd_hbm.at[pl.ds(off, c)])

    return k(xw, pos16, src, dst)


def _tc_message(xwg, ps, pd, W1, b1, W2, b2, Wy16):
    e, d = xwg.shape
    b = 512
    s3 = np.float32(np.sqrt(3.0))

    def body(xwg_ref, ps_ref, pd_ref, w1_ref, b1_ref, w2_ref, b2_ref,
             wy_ref, o_ref):
        dv = pd_ref[...] - ps_ref[...]
        len2 = jnp.sum(dv * dv, axis=1, keepdims=True)
        ln = jnp.maximum(jnp.sqrt(len2), 1e-8)
        dirv = dv / ln
        xx = dirv[:, 0:1]
        yy = dirv[:, 1:2]
        zz = dirv[:, 2:3]
        Y = jnp.concatenate(
            [
                jnp.ones_like(xx),
                xx, yy, zz,
                s3 * xx * yy,
                s3 * yy * zz,
                0.5 * (3.0 * zz * zz - 1.0),
                s3 * xx * zz,
                (s3 / 2.0) * (xx * xx - yy * yy),
                jnp.zeros((b, 7), jnp.float32),
            ],
            axis=1,
        )
        yw = jnp.dot(Y, wy_ref[...], precision=_H)
        h = jax.nn.silu(ln * w1_ref[...] + b1_ref[...])
        w = jnp.dot(h, w2_ref[...], precision=_H) + b2_ref[...]
        o_ref[...] = xwg_ref[...] * yw * w

    return pl.pallas_call(
        body,
        grid=(e // b,),
        in_specs=[
            pl.BlockSpec((b, d), lambda i: (i, 0)),
            pl.BlockSpec((b, 16), lambda i: (i, 0)),
            pl.BlockSpec((b, 16), lambda i: (i, 0)),
            pl.BlockSpec((1, 64), lambda i: (0, 0)),
            pl.BlockSpec((1, 64), lambda i: (0, 0)),
            pl.BlockSpec((64, 128), lambda i: (0, 0)),
            pl.BlockSpec((1, 128), lambda i: (0, 0)),
            pl.BlockSpec((16, 128), lambda i: (0, 0)),
        ],
        out_specs=pl.BlockSpec((b, d), lambda i: (i, 0)),
        out_shape=jax.ShapeDtypeStruct((e, d), jnp.float32),
    )(xwg, ps, pd, W1, b1, W2, b2, Wy16)


def _sc_scatter(m, dst, n):
    e, d = m.shape
    c = 200
    e_per_core = e // _NC
    per_sub = e_per_core // _NS
    steps = per_sub // c
    wb_rows = 1000
    zb = 40
    mesh = plsc.VectorSubcoreMesh(core_axis_name="c", subcore_axis_name="s")

    @functools.partial(
        pl.kernel,
        out_type=jax.ShapeDtypeStruct((_NC, n, d), jnp.float32),
        mesh=mesh,
        scratch_types=[
            pltpu.VMEM((c,), jnp.int32),
            pltpu.VMEM((c, d), jnp.float32),
            pltpu.VMEM((zb, d), jnp.float32),
            pltpu.VMEM_SHARED((n, d), jnp.float32),
        ],
    )
    def k(m_hbm, dst_hbm, out_hbm, idx_v, rows_v, zero_v, acc_sh):
        cid = lax.axis_index("c")
        sid = lax.axis_index("s")
        zvec = jnp.zeros((16,), jnp.float32)

        @pl.loop(0, zb)
        def _(r):
            @pl.loop(0, d // 16)
            def _(j):
                zero_v.at[r, pl.ds(j * 16, 16)][...] = zvec

        @pl.when(sid < n // wb_rows)
        def _():
            @pl.loop(0, wb_rows // zb)
            def _(bk):
                pltpu.sync_copy(zero_v,
                                acc_sh.at[pl.ds(sid * wb_rows + bk * zb, zb)])

        plsc.subcore_barrier()

        base = cid * e_per_core + sid * per_sub

        @pl.loop(0, steps)
        def _(i):
            off = base + i * c
            pltpu.sync_copy(dst_hbm.at[pl.ds(off, c)], idx_v)
            pltpu.sync_copy(m_hbm.at[pl.ds(off, c)], rows_v)
            pltpu.sync_copy(rows_v, acc_sh.at[idx_v], add=True)

        plsc.subcore_barrier()

        @pl.when(sid < n // wb_rows)
        def _():
            pltpu.sync_copy(acc_sh.at[pl.ds(sid * wb_rows, wb_rows)],
                            out_hbm.at[cid, pl.ds(sid * wb_rows, wb_rows)])

    return k(m, dst)


def _tc_head(parts, Ws, Wns, Wg):
    _, n, d = parts.shape
    b = 1000

    def body(p_ref, ws_ref, wns_ref, wg_ref, o_ref):
        out = p_ref[0] + p_ref[1]
        s = jax.nn.silu(jnp.dot(out, ws_ref[...], precision=_H))
        ns = jnp.dot(out, wns_ref[...], precision=_H)
        g = jax.nn.sigmoid(jnp.dot(out, wg_ref[...], precision=_H))
        i0 = lax.broadcasted_iota(jnp.int32, (32, 96), 0)
        i1 = lax.broadcasted_iota(jnp.int32, (32, 96), 1)
        rep = (i0 == i1 // 3).astype(jnp.float32)
        gr = jnp.dot(g, rep, precision=_H)
        o_ref[...] = jnp.concatenate([s, gr * ns], axis=1)

    return pl.pallas_call(
        body,
        grid=(n // b,),
        in_specs=[
            pl.BlockSpec((2, b, d), lambda i: (0, i, 0)),
            pl.BlockSpec((128, 32), lambda i: (0, 0)),
            pl.BlockSpec((128, 96), lambda i: (0, 0)),
            pl.BlockSpec((128, 32), lambda i: (0, 0)),
        ],
        out_specs=pl.BlockSpec((b, d), lambda i: (i, 0)),
        out_shape=jax.ShapeDtypeStruct((n, d), jnp.float32),
    )(parts, Ws, Wns, Wg)


def kernel(x, edge_index, pos, W1, b1, W2, b2, Wx, Wy, Ws, Wns, Wg):
    n = x.shape[0]
    src = edge_index[0]
    dst = edge_index[1]
    pos16 = jnp.pad(pos, ((0, 0), (0, 13)))
    Wy16 = jnp.pad(Wy, ((0, 7), (0, 0)))
    b1r = b1.reshape(1, -1)
    b2r = b2.reshape(1, -1)

    xw = _tc_matmul(x, Wx)
    xwg, ps, pd = _sc_gather_all(xw, pos16, src, dst)
    m = _tc_message(xwg, ps, pd, W1, b1r, W2, b2r, Wy16)
    parts = _sc_scatter(m, dst, n)
    return _tc_head(parts, Ws, Wns, Wg)


# speedup vs baseline: 4.2779x; 1.7991x over previous
import functools

import jax
import jax.numpy as jnp
import numpy as np
from jax import lax
from jax.experimental import pallas as pl
from jax.experimental.pallas import tpu as pltpu
from jax.experimental.pallas import tpu_sc as plsc

_NC = 2
_NS = 16
_NW = _NC * _NS
_H = jax.lax.Precision.HIGHEST


def _tc_matmul(x, Wx):
    n, d = x.shape
    b = 1000

    def body(x_ref, w_ref, o_ref):
        o_ref[...] = jnp.dot(x_ref[...], w_ref[...], precision=_H)

    return pl.pallas_call(
        body,
        grid=(n // b,),
        in_specs=[
            pl.BlockSpec((b, d), lambda i: (i, 0)),
            pl.BlockSpec(Wx.shape, lambda i: (0, 0)),
        ],
        out_specs=pl.BlockSpec((b, Wx.shape[1]), lambda i: (i, 0)),
        out_shape=jax.ShapeDtypeStruct((n, Wx.shape[1]), jnp.float32),
    )(x, Wx)


def _sc_gather_rows(xw, src):
    e = src.shape[0]
    d = xw.shape[1]
    c = 400
    per_w = e // _NW
    steps = per_w // c
    mesh = plsc.VectorSubcoreMesh(core_axis_name="c", subcore_axis_name="s")

    @functools.partial(
        pl.kernel,
        out_type=jax.ShapeDtypeStruct((e, d), jnp.float32),
        mesh=mesh,
        scratch_types=[
            pltpu.VMEM((c,), jnp.int32),
            pltpu.VMEM((c, d), jnp.float32),
            pltpu.SemaphoreType.DMA,
        ],
    )
    def k(xw_hbm, src_hbm, xwg_hbm, idx_v, rows_v, sem):
        wid = lax.axis_index("s") * _NC + lax.axis_index("c")
        base = wid * per_w

        @pl.loop(0, steps)
        def _(i):
            off = base + i * c
            pltpu.sync_copy(src_hbm.at[pl.ds(off, c)], idx_v)
            pltpu.async_copy(xw_hbm.at[idx_v], rows_v, sem).wait()
            pltpu.sync_copy(rows_v, xwg_hbm.at[pl.ds(off, c)])

    return k(xw, src)


def _sc_gather_dvec(pos_flat, src, dst):
    e = src.shape[0]
    npts8 = pos_flat.shape[0]
    c = 400
    per_w = e // _NW
    steps = per_w // c
    mesh = plsc.VectorSubcoreMesh(core_axis_name="c", subcore_axis_name="s")

    @functools.partial(
        pl.kernel,
        out_type=jax.ShapeDtypeStruct((3, _NW, per_w), jnp.float32),
        mesh=mesh,
        scratch_types=[
            pltpu.VMEM((npts8,), jnp.float32),
            pltpu.VMEM((c,), jnp.int32),
            pltpu.VMEM((c,), jnp.int32),
            pltpu.VMEM((c,), jnp.float32),
            pltpu.VMEM((c,), jnp.float32),
            pltpu.VMEM((c,), jnp.float32),
        ],
        compiler_params=pltpu.CompilerParams(use_tc_tiling_on_sc=False,
                                             needs_layout_passes=False),
    )
    def k(pos_hbm, src_hbm, dst_hbm, dv_hbm, pos_v, idxs_v, idxd_v,
          dx_v, dy_v, dz_v):
        wid = lax.axis_index("s") * _NC + lax.axis_index("c")
        base = wid * per_w
        pltpu.sync_copy(pos_hbm, pos_v)

        @pl.loop(0, steps)
        def _(i):
            off = base + i * c
            pltpu.sync_copy(src_hbm.at[pl.ds(off, c)], idxs_v)
            pltpu.sync_copy(dst_hbm.at[pl.ds(off, c)], idxd_v)

            @pl.loop(0, c // 16)
            def _(g):
                sl = pl.ds(g * 16, 16)
                s8 = idxs_v[sl] * 8
                d8 = idxd_v[sl] * 8
                dx_v.at[sl][...] = (plsc.load_gather(pos_v, [d8])
                                    - plsc.load_gather(pos_v, [s8]))
                dy_v.at[sl][...] = (plsc.load_gather(pos_v, [d8 + 1])
                                    - plsc.load_gather(pos_v, [s8 + 1]))
                dz_v.at[sl][...] = (plsc.load_gather(pos_v, [d8 + 2])
                                    - plsc.load_gather(pos_v, [s8 + 2]))

            pltpu.sync_copy(dx_v, dv_hbm.at[0, wid, pl.ds(i * c, c)])
            pltpu.sync_copy(dy_v, dv_hbm.at[1, wid, pl.ds(i * c, c)])
            pltpu.sync_copy(dz_v, dv_hbm.at[2, wid, pl.ds(i * c, c)])

    return k(pos_flat, src, dst)


def _tc_message(xwg, dxa, dya, dza, W1T, b1T, W2, b2, Wy16):
    e, d = xwg.shape
    nb, _, b = dxa.shape
    s3 = np.float32(np.sqrt(3.0))
    dn = (((0,), (0,)), ((), ()))

    def body(xwg_ref, dx_ref, dy_ref, dz_ref, w1_ref, b1_ref, w2_ref,
             b2_ref, wy_ref, o_ref):
        dx = dx_ref[0]
        dy = dy_ref[0]
        dz = dz_ref[0]
        d2 = dx * dx + dy * dy + dz * dz
        ln = jnp.maximum(jnp.sqrt(d2), 1e-8)
        inv = 1.0 / ln
        ex = dx * inv
        ey = dy * inv
        ez = dz * inv
        Yl = jnp.concatenate(
            [
                jnp.ones_like(ex),
                ex, ey, ez,
                s3 * ex * ey,
                s3 * ey * ez,
                0.5 * (3.0 * ez * ez - 1.0),
                s3 * ex * ez,
                (s3 / 2.0) * (ex * ex - ey * ey),
                jnp.zeros((7, b), jnp.float32),
            ],
            axis=0,
        )
        yw = lax.dot_general(Yl, wy_ref[...], dn, precision=_H)
        hl = jax.nn.silu(w1_ref[...] * ln + b1_ref[...])
        w = lax.dot_general(hl, w2_ref[...], dn, precision=_H) + b2_ref[...]
        o_ref[...] = xwg_ref[...] * (yw * w)

    return pl.pallas_call(
        body,
        grid=(nb,),
        in_specs=[
            pl.BlockSpec((b, d), lambda i: (i, 0)),
            pl.BlockSpec((1, 1, b), lambda i: (i, 0, 0)),
            pl.BlockSpec((1, 1, b), lambda i: (i, 0, 0)),
            pl.BlockSpec((1, 1, b), lambda i: (i, 0, 0)),
            pl.BlockSpec((64, 1), lambda i: (0, 0)),
            pl.BlockSpec((64, 1), lambda i: (0, 0)),
            pl.BlockSpec((64, 128), lambda i: (0, 0)),
            pl.BlockSpec((1, 128), lambda i: (0, 0)),
            pl.BlockSpec((16, 128), lambda i: (0, 0)),
        ],
        out_specs=pl.BlockSpec((b, d), lambda i: (i, 0)),
        out_shape=jax.ShapeDtypeStruct((e, d), jnp.float32),
    )(xwg, dxa, dya, dza, W1T, b1T, W2, b2, Wy16)


def _sc_scatter(m, dst, n):
    e, d = m.shape
    c = 200
    e_per_core = e // _NC
    per_sub = e_per_core // _NS
    steps = per_sub // c
    wb_rows = 1000
    zb = 40
    mesh = plsc.VectorSubcoreMesh(core_axis_name="c", subcore_axis_name="s")

    @functools.partial(
        pl.kernel,
        out_type=jax.ShapeDtypeStruct((_NC, n, d), jnp.float32),
        mesh=mesh,
        scratch_types=[
            pltpu.VMEM((c,), jnp.int32),
            pltpu.VMEM((c, d), jnp.float32),
            pltpu.VMEM((zb, d), jnp.float32),
            pltpu.VMEM_SHARED((n, d), jnp.float32),
        ],
    )
    def k(m_hbm, dst_hbm, out_hbm, idx_v, rows_v, zero_v, acc_sh):
        cid = lax.axis_index("c")
        sid = lax.axis_index("s")
        zvec = jnp.zeros((16,), jnp.float32)

        @pl.loop(0, zb)
        def _(r):
            @pl.loop(0, d // 16)
            def _(j):
                zero_v.at[r, pl.ds(j * 16, 16)][...] = zvec

        @pl.when(sid < n // wb_rows)
        def _():
            @pl.loop(0, wb_rows // zb)
            def _(bk):
                pltpu.sync_copy(zero_v,
                                acc_sh.at[pl.ds(sid * wb_rows + bk * zb, zb)])

        plsc.subcore_barrier()

        base = cid * e_per_core + sid * per_sub

        @pl.loop(0, steps)
        def _(i):
            off = base + i * c
            pltpu.sync_copy(dst_hbm.at[pl.ds(off, c)], idx_v)
            pltpu.sync_copy(m_hbm.at[pl.ds(off, c)], rows_v)
            pltpu.sync_copy(rows_v, acc_sh.at[idx_v], add=True)

        plsc.subcore_barrier()

        @pl.when(sid < n // wb_rows)
        def _():
            pltpu.sync_copy(acc_sh.at[pl.ds(sid * wb_rows, wb_rows)],
                            out_hbm.at[cid, pl.ds(sid * wb_rows, wb_rows)])

    return k(m, dst)


def _tc_head(parts, Ws, Wns, Wg):
    _, n, d = parts.shape
    b = 1000

    def body(p_ref, ws_ref, wns_ref, wg_ref, o_ref):
        out = p_ref[0] + p_ref[1]
        s = jax.nn.silu(jnp.dot(out, ws_ref[...], precision=_H))
        ns = jnp.dot(out, wns_ref[...], precision=_H)
        g = jax.nn.sigmoid(jnp.dot(out, wg_ref[...], precision=_H))
        i0 = lax.broadcasted_iota(jnp.int32, (32, 96), 0)
        i1 = lax.broadcasted_iota(jnp.int32, (32, 96), 1)
        rep = (i0 == i1 // 3).astype(jnp.float32)
        gr = jnp.dot(g, rep, precision=_H)
        o_ref[...] = jnp.concatenate([s, gr * ns], axis=1)

    return pl.pallas_call(
        body,
        grid=(n // b,),
        in_specs=[
            pl.BlockSpec((2, b, d), lambda i: (0, i, 0)),
            pl.BlockSpec((128, 32), lambda i: (0, 0)),
            pl.BlockSpec((128, 96), lambda i: (0, 0)),
            pl.BlockSpec((128, 32), lambda i: (0, 0)),
        ],
        out_specs=pl.BlockSpec((b, d), lambda i: (i, 0)),
        out_shape=jax.ShapeDtypeStruct((n, d), jnp.float32),
    )(parts, Ws, Wns, Wg)


def kernel(x, edge_index, pos, W1, b1, W2, b2, Wx, Wy, Ws, Wns, Wg):
    n = x.shape[0]
    e = edge_index.shape[1]
    be = 1280
    src = edge_index[0]
    dst = edge_index[1]
    pos_flat = jnp.pad(pos, ((0, 0), (0, 5))).reshape(-1)
    Wy16 = jnp.pad(Wy, ((0, 7), (0, 0)))
    W1T = W1.reshape(-1, 1)
    b1T = b1.reshape(-1, 1)
    b2r = b2.reshape(1, -1)

    xw = _tc_matmul(x, Wx)
    dv = _sc_gather_dvec(pos_flat, src, dst)
    xwg = _sc_gather_rows(xw, src)
    dxa = dv[0].reshape(e // be, 1, be)
    dya = dv[1].reshape(e // be, 1, be)
    dza = dv[2].reshape(e // be, 1, be)
    m = _tc_message(xwg, dxa, dya, dza, W1T, b1T, W2, b2r, Wy16)
    parts = _sc_scatter(m, dst, n)
    return _tc_head(parts, Ws, Wns, Wg)


# speedup vs baseline: 5.5885x; 1.3064x over previous
import functools

import jax
import jax.numpy as jnp
import numpy as np
from jax import lax
from jax.experimental import pallas as pl
from jax.experimental.pallas import tpu as pltpu
from jax.experimental.pallas import tpu_sc as plsc

_NC = 2
_NS = 16
_NW = _NC * _NS
_H = jax.lax.Precision.HIGHEST


def _tc_matmul(x, Wx):
    n, d = x.shape
    b = 1000

    def body(x_ref, w_ref, o_ref):
        o_ref[...] = jnp.dot(x_ref[...], w_ref[...], precision=_H)

    return pl.pallas_call(
        body,
        grid=(n // b,),
        in_specs=[
            pl.BlockSpec((b, d), lambda i: (i, 0)),
            pl.BlockSpec(Wx.shape, lambda i: (0, 0)),
        ],
        out_specs=pl.BlockSpec((b, Wx.shape[1]), lambda i: (i, 0)),
        out_shape=jax.ShapeDtypeStruct((n, Wx.shape[1]), jnp.float32),
    )(x, Wx)


def _sc_gather_rows(xw, src):
    e = src.shape[0]
    d = xw.shape[1]
    c = 400
    per_w = e // _NW
    steps = per_w // c
    mesh = plsc.VectorSubcoreMesh(core_axis_name="c", subcore_axis_name="s")

    @functools.partial(
        pl.kernel,
        out_type=jax.ShapeDtypeStruct((e, d), jnp.float32),
        mesh=mesh,
        scratch_types=[
            pltpu.VMEM((c,), jnp.int32),
            pltpu.VMEM((c, d), jnp.float32),
            pltpu.SemaphoreType.DMA,
        ],
    )
    def k(xw_hbm, src_hbm, xwg_hbm, idx_v, rows_v, sem):
        wid = lax.axis_index("s") * _NC + lax.axis_index("c")
        base = wid * per_w

        @pl.loop(0, steps)
        def _(i):
            off = base + i * c
            pltpu.sync_copy(src_hbm.at[pl.ds(off, c)], idx_v)
            pltpu.async_copy(xw_hbm.at[idx_v], rows_v, sem).wait()
            pltpu.sync_copy(rows_v, xwg_hbm.at[pl.ds(off, c)])

    return k(xw, src)


def _sc_gather_dvec(pos_flat, src, dst):
    e = src.shape[0]
    npts8 = pos_flat.shape[0]
    c = 400
    per_w = e // _NW
    steps = per_w // c
    mesh = plsc.VectorSubcoreMesh(core_axis_name="c", subcore_axis_name="s")

    @functools.partial(
        pl.kernel,
        out_type=jax.ShapeDtypeStruct((3, _NW, per_w), jnp.float32),
        mesh=mesh,
        scratch_types=[
            pltpu.VMEM((npts8,), jnp.float32),
            pltpu.VMEM((c,), jnp.int32),
            pltpu.VMEM((c,), jnp.int32),
            pltpu.VMEM((c,), jnp.float32),
            pltpu.VMEM((c,), jnp.float32),
            pltpu.VMEM((c,), jnp.float32),
        ],
        compiler_params=pltpu.CompilerParams(use_tc_tiling_on_sc=False,
                                             needs_layout_passes=False),
    )
    def k(pos_hbm, src_hbm, dst_hbm, dv_hbm, pos_v, idxs_v, idxd_v,
          dx_v, dy_v, dz_v):
        wid = lax.axis_index("s") * _NC + lax.axis_index("c")
        base = wid * per_w
        pltpu.sync_copy(pos_hbm, pos_v)

        @pl.loop(0, steps)
        def _(i):
            off = base + i * c
            pltpu.sync_copy(src_hbm.at[pl.ds(off, c)], idxs_v)
            pltpu.sync_copy(dst_hbm.at[pl.ds(off, c)], idxd_v)

            @pl.loop(0, c // 16)
            def _(g):
                sl = pl.ds(g * 16, 16)
                s8 = idxs_v[sl] * 8
                d8 = idxd_v[sl] * 8
                dx_v.at[sl][...] = (plsc.load_gather(pos_v, [d8])
                                    - plsc.load_gather(pos_v, [s8]))
                dy_v.at[sl][...] = (plsc.load_gather(pos_v, [d8 + 1])
                                    - plsc.load_gather(pos_v, [s8 + 1]))
                dz_v.at[sl][...] = (plsc.load_gather(pos_v, [d8 + 2])
                                    - plsc.load_gather(pos_v, [s8 + 2]))

            pltpu.sync_copy(dx_v, dv_hbm.at[0, wid, pl.ds(i * c, c)])
            pltpu.sync_copy(dy_v, dv_hbm.at[1, wid, pl.ds(i * c, c)])
            pltpu.sync_copy(dz_v, dv_hbm.at[2, wid, pl.ds(i * c, c)])

    return k(pos_flat, src, dst)


def _tc_message(xwg, dxa, dya, dza, W1T, b1T, W2, b2, Wy16):
    e, d = xwg.shape
    nb, _, b = dxa.shape
    s3 = np.float32(np.sqrt(3.0))
    dn = (((0,), (0,)), ((), ()))

    def body(xwg_ref, dx_ref, dy_ref, dz_ref, w1_ref, b1_ref, w2_ref,
             b2_ref, wy_ref, o_ref):
        dx = dx_ref[0]
        dy = dy_ref[0]
        dz = dz_ref[0]
        d2 = dx * dx + dy * dy + dz * dz
        ln = jnp.maximum(jnp.sqrt(d2), 1e-8)
        inv = 1.0 / ln
        ex = dx * inv
        ey = dy * inv
        ez = dz * inv
        Yl = jnp.concatenate(
            [
                jnp.ones_like(ex),
                ex, ey, ez,
                s3 * ex * ey,
                s3 * ey * ez,
                0.5 * (3.0 * ez * ez - 1.0),
                s3 * ex * ez,
                (s3 / 2.0) * (ex * ex - ey * ey),
                jnp.zeros((7, b), jnp.float32),
            ],
            axis=0,
        )
        yw = lax.dot_general(Yl, wy_ref[...], dn, precision=None)
        hl = jax.nn.silu(w1_ref[...] * ln + b1_ref[...])
        w = lax.dot_general(hl, w2_ref[...], dn, precision=None) + b2_ref[...]
        o_ref[...] = xwg_ref[...] * (yw * w)

    return pl.pallas_call(
        body,
        grid=(nb,),
        in_specs=[
            pl.BlockSpec((b, d), lambda i: (i, 0)),
            pl.BlockSpec((1, 1, b), lambda i: (i, 0, 0)),
            pl.BlockSpec((1, 1, b), lambda i: (i, 0, 0)),
            pl.BlockSpec((1, 1, b), lambda i: (i, 0, 0)),
            pl.BlockSpec((64, 1), lambda i: (0, 0)),
            pl.BlockSpec((64, 1), lambda i: (0, 0)),
            pl.BlockSpec((64, 128), lambda i: (0, 0)),
            pl.BlockSpec((1, 128), lambda i: (0, 0)),
            pl.BlockSpec((16, 128), lambda i: (0, 0)),
        ],
        out_specs=pl.BlockSpec((b, d), lambda i: (i, 0)),
        out_shape=jax.ShapeDtypeStruct((e, d), jnp.float32),
    )(xwg, dxa, dya, dza, W1T, b1T, W2, b2, Wy16)


def _sc_scatter(m, dst, n):
    e, d = m.shape
    c = 200
    e_per_core = e // _NC
    per_sub = e_per_core // _NS
    steps = per_sub // c
    wb_rows = 1000
    zb = 40
    mesh = plsc.VectorSubcoreMesh(core_axis_name="c", subcore_axis_name="s")

    @functools.partial(
        pl.kernel,
        out_type=jax.ShapeDtypeStruct((_NC, n, d), jnp.float32),
        mesh=mesh,
        scratch_types=[
            pltpu.VMEM((c,), jnp.int32),
            pltpu.VMEM((c, d), jnp.float32),
            pltpu.VMEM((zb, d), jnp.float32),
            pltpu.VMEM_SHARED((n, d), jnp.float32),
        ],
    )
    def k(m_hbm, dst_hbm, out_hbm, idx_v, rows_v, zero_v, acc_sh):
        cid = lax.axis_index("c")
        sid = lax.axis_index("s")
        zvec = jnp.zeros((16,), jnp.float32)

        @pl.loop(0, zb)
        def _(r):
            @pl.loop(0, d // 16)
            def _(j):
                zero_v.at[r, pl.ds(j * 16, 16)][...] = zvec

        @pl.when(sid < n // wb_rows)
        def _():
            @pl.loop(0, wb_rows // zb)
            def _(bk):
                pltpu.sync_copy(zero_v,
                                acc_sh.at[pl.ds(sid * wb_rows + bk * zb, zb)])

        plsc.subcore_barrier()

        base = cid * e_per_core + sid * per_sub

        @pl.loop(0, steps)
        def _(i):
            off = base + i * c
            pltpu.sync_copy(dst_hbm.at[pl.ds(off, c)], idx_v)
            pltpu.sync_copy(m_hbm.at[pl.ds(off, c)], rows_v)
            pltpu.sync_copy(rows_v, acc_sh.at[idx_v], add=True)

        plsc.subcore_barrier()

        @pl.when(sid < n // wb_rows)
        def _():
            pltpu.sync_copy(acc_sh.at[pl.ds(sid * wb_rows, wb_rows)],
                            out_hbm.at[cid, pl.ds(sid * wb_rows, wb_rows)])

    return k(m, dst)


def _tc_head(parts, Ws, Wns, Wg):
    _, n, d = parts.shape
    b = 1000

    def body(p_ref, ws_ref, wns_ref, wg_ref, o_ref):
        out = p_ref[0] + p_ref[1]
        s = jax.nn.silu(jnp.dot(out, ws_ref[...], precision=_H))
        ns = jnp.dot(out, wns_ref[...], precision=_H)
        g = jax.nn.sigmoid(jnp.dot(out, wg_ref[...], precision=_H))
        i0 = lax.broadcasted_iota(jnp.int32, (32, 96), 0)
        i1 = lax.broadcasted_iota(jnp.int32, (32, 96), 1)
        rep = (i0 == i1 // 3).astype(jnp.float32)
        gr = jnp.dot(g, rep, precision=_H)
        o_ref[...] = jnp.concatenate([s, gr * ns], axis=1)

    return pl.pallas_call(
        body,
        grid=(n // b,),
        in_specs=[
            pl.BlockSpec((2, b, d), lambda i: (0, i, 0)),
            pl.BlockSpec((128, 32), lambda i: (0, 0)),
            pl.BlockSpec((128, 96), lambda i: (0, 0)),
            pl.BlockSpec((128, 32), lambda i: (0, 0)),
        ],
        out_specs=pl.BlockSpec((b, d), lambda i: (i, 0)),
        out_shape=jax.ShapeDtypeStruct((n, d), jnp.float32),
    )(parts, Ws, Wns, Wg)


def kernel(x, edge_index, pos, W1, b1, W2, b2, Wx, Wy, Ws, Wns, Wg):
    n = x.shape[0]
    e = edge_index.shape[1]
    be = 2560
    src = edge_index[0]
    dst = edge_index[1]
    pos_flat = jnp.pad(pos, ((0, 0), (0, 5))).reshape(-1)
    Wy16 = jnp.pad(Wy, ((0, 7), (0, 0)))
    W1T = W1.reshape(-1, 1)
    b1T = b1.reshape(-1, 1)
    b2r = b2.reshape(1, -1)

    xw = _tc_matmul(x, Wx)
    dv = _sc_gather_dvec(pos_flat, src, dst)
    xwg = _sc_gather_rows(xw, src)
    dxa = dv[0].reshape(e // be, 1, be)
    dya = dv[1].reshape(e // be, 1, be)
    dza = dv[2].reshape(e // be, 1, be)
    m = _tc_message(xwg, dxa, dya, dza, W1T, b1T, W2, b2r, Wy16)
    parts = _sc_scatter(m, dst, n)
    return _tc_head(parts, Ws, Wns, Wg)
